# baseline (speedup 1.0000x reference)
import functools

import jax
import jax.numpy as jnp
from jax import lax
from jax.experimental import pallas as pl
from jax.experimental.pallas import tpu as pltpu
from jax.experimental.pallas import tpu_sc as plsc

B, A, N, G, F = 2, 10000, 16, 16, 128
TA = 400
TM = 2000
SLICES = (5200, 4800)

SC_NC, SC_NS = 2, 16
SC_NW = SC_NC * SC_NS
SC_C = 40
SC_NB = 5


def _make_sc_gather(R):
    PW = R // SC_NW
    NCH = PW // SC_C
    assert PW % SC_C == 0 and NCH % SC_NB == 0 and PW % 8 == 0

    def body(x_hbm, gidx_hbm, out_hbm, idx_all, rows, *sems):
        gsem = sems[:SC_NB]
        wsem = sems[SC_NB:]
        wid = lax.axis_index("s") * SC_NC + lax.axis_index("c")
        base = wid * PW
        pltpu.sync_copy(gidx_hbm.at[pl.ds(base, PW)], idx_all)

        def g_desc(k, b):
            return pltpu.make_async_copy(
                x_hbm.at[idx_all.at[pl.ds(k * SC_C, SC_C)]], rows.at[b], gsem[b])

        def w_desc(k, b):
            return pltpu.make_async_copy(
                rows.at[b], out_hbm.at[pl.ds(base + k * SC_C, SC_C)], wsem[b])

        for b in range(SC_NB):
            g_desc(b, b).start()

        def outer(t, carry):
            for b in range(SC_NB):
                k = SC_NB * t + b
                g_desc(k, b).wait()
                w_desc(k, b).start()
                w_desc(k, b).wait()
                g_desc(k + SC_NB, b).start()
            return carry

        lax.fori_loop(0, NCH // SC_NB - 1, outer, 0)
        for b in range(SC_NB):
            k = NCH - SC_NB + b
            g_desc(k, b).wait()
            w_desc(k, b).start()
            w_desc(k, b).wait()

    return pl.kernel(
        body,
        out_type=jax.ShapeDtypeStruct((R, F), jnp.float32),
        mesh=plsc.VectorSubcoreMesh(core_axis_name="c", subcore_axis_name="s"),
        scratch_types=(
            [pltpu.VMEM((PW,), jnp.int32),
             pltpu.VMEM((SC_NB, SC_C, F), jnp.float32)]
            + [pltpu.SemaphoreType.DMA] * (2 * SC_NB)
        ),
    )


_SC_GATHERS = {R: _make_sc_gather(R) for R in (N * SLICES[0], N * SLICES[1])}


def _xw_body(f_ref, w_ref, o_ref):
    o_ref[...] = jnp.dot(f_ref[...], w_ref[...], preferred_element_type=jnp.float32)


def _project_x(features2, W0):
    M = features2.shape[0]
    return pl.pallas_call(
        _xw_body,
        grid=(M // TM,),
        in_specs=[
            pl.BlockSpec((TM, F), lambda i: (i, 0)),
            pl.BlockSpec((F, F), lambda i: (0, 0)),
        ],
        out_specs=pl.BlockSpec((TM, F), lambda i: (i, 0)),
        out_shape=jax.ShapeDtypeStruct((M, F), jnp.float32),
    )(features2, W0)


def _fused_body(nbr_ref, rbf_ref, wf1_ref, bf1_ref, wf2_ref, bf2_ref, v_ref,
                emat_ref, w1_ref, b1_ref, w2_ref, b2_ref, out_ref, attn_ref,
                filt_scr):
    cols = []
    for n in range(N):
        r_n = rbf_ref[:, n * G:(n + 1) * G]
        h_n = jnp.tanh(jnp.dot(r_n, wf1_ref[...], preferred_element_type=jnp.float32)
                       + bf1_ref[...])
        f_n = jnp.dot(h_n, wf2_ref[...], preferred_element_type=jnp.float32) + bf2_ref[...]
        filt_n = nbr_ref[n] * f_n
        filt_scr[n] = filt_n
        cols.append(jnp.dot(filt_n, v_ref[...], preferred_element_type=jnp.float32))
    logits = jnp.concatenate(cols, axis=1)
    m = jnp.max(logits, axis=-1, keepdims=True)
    e = jnp.exp(logits - m)
    s = jnp.sum(e, axis=-1, keepdims=True)
    attn = e / s
    attn_ref[...] = attn
    bc = jnp.dot(attn, emat_ref[...], preferred_element_type=jnp.float32)
    conv = jnp.zeros((TA, F), dtype=jnp.float32)
    for n in range(N):
        conv = conv + bc[:, n * F:(n + 1) * F] * filt_scr[n]
    t = jnp.tanh(jnp.dot(conv, w1_ref[...], preferred_element_type=jnp.float32)
                 + b1_ref[...])
    out_ref[...] = jnp.dot(t, w2_ref[...], preferred_element_type=jnp.float32) + b2_ref[...]


def _fused_one(nbr3, rbf2, Wf1, bf1, Wf2, bf2, v, emat, W1, b1, W2, b2):
    AL = rbf2.shape[0]
    full = lambda j: (0, 0)
    out_shape = (
        jax.ShapeDtypeStruct((AL, F), jnp.float32),
        jax.ShapeDtypeStruct((AL, N), jnp.float32),
    )
    return pl.pallas_call(
        _fused_body,
        grid=(AL // TA,),
        in_specs=[
            pl.BlockSpec((N, TA, F), lambda j: (0, j, 0)),
            pl.BlockSpec((TA, N * G), lambda j: (j, 0)),
            pl.BlockSpec((G, F), full),
            pl.BlockSpec((1, F), full),
            pl.BlockSpec((F, F), full),
            pl.BlockSpec((1, F), full),
            pl.BlockSpec((F, 1), full),
            pl.BlockSpec((N, N * F), full),
            pl.BlockSpec((F, F), full),
            pl.BlockSpec((1, F), full),
            pl.BlockSpec((F, F), full),
            pl.BlockSpec((1, F), full),
        ],
        out_specs=(
            pl.BlockSpec((TA, F), lambda j: (j, 0)),
            pl.BlockSpec((TA, N), lambda j: (j, 0)),
        ),
        out_shape=out_shape,
        scratch_shapes=[
            pltpu.VMEM((N, TA, F), jnp.float32),
        ],
    )(nbr3, rbf2, Wf1, bf1, Wf2, bf2, v, emat, W1, b1, W2, b2)


def kernel(features, rbf_expansion, neighbor_list, W0, Wf1, bf1, Wf2, bf2,
           nbr_filter, W1, b1, W2, b2):
    bf1r, bf2r = bf1.reshape(1, F), bf2.reshape(1, F)
    b1r, b2r = b1.reshape(1, F), b2.reshape(1, F)
    emat = jnp.repeat(jnp.eye(N, dtype=jnp.float32), F, axis=1)
    outs, attns = [], []
    for b in range(B):
        x2 = _project_x(features[b], W0)
        nlT = jnp.swapaxes(neighbor_list[b], 0, 1)
        o_parts, at_parts = [], []
        a0 = 0
        for al in SLICES:
            gidx = nlT[:, a0:a0 + al].reshape(-1)
            nbr3 = _SC_GATHERS[N * al](x2, gidx).reshape(N, al, F)
            o, at = _fused_one(
                nbr3, rbf_expansion[b, a0:a0 + al].reshape(al, N * G),
                Wf1, bf1r, Wf2, bf2r, nbr_filter, emat, W1, b1r, W2, b2r)
            o_parts.append(o)
            at_parts.append(at)
            a0 += al
        outs.append(jnp.concatenate(o_parts, axis=0))
        attns.append(jnp.concatenate(at_parts, axis=0))
    return (jnp.stack(outs), jnp.stack(attns))


# speedup vs baseline: 1.1304x; 1.1304x over previous
import functools

import jax
import jax.numpy as jnp
from jax import lax
from jax.experimental import pallas as pl
from jax.experimental.pallas import tpu as pltpu
from jax.experimental.pallas import tpu_sc as plsc

B, A, N, G, F = 2, 10000, 16, 16, 128
TA = 1000
TM = 2000

SC_NC, SC_NS = 2, 16
SC_NW = SC_NC * SC_NS
SC_R = A * N
SC_PW = SC_R // SC_NW
SC_C = 40
SC_NB = 5
SC_NCH = SC_PW // SC_C


def _sc_gather_body(x_hbm, gidx_hbm, out_hbm, idx_all, rows, *sems):
    gsem = sems[:SC_NB]
    wsem = sems[SC_NB:]
    wid = lax.axis_index("s") * SC_NC + lax.axis_index("c")
    base = wid * SC_PW
    pltpu.sync_copy(gidx_hbm.at[pl.ds(base, SC_PW)], idx_all)

    def g_desc(k, b):
        return pltpu.make_async_copy(
            x_hbm.at[idx_all.at[pl.ds(k * SC_C, SC_C)]], rows.at[b], gsem[b])

    def w_desc(k, b):
        return pltpu.make_async_copy(
            rows.at[b], out_hbm.at[pl.ds(base + k * SC_C, SC_C)], wsem[b])

    for b in range(SC_NB):
        g_desc(b, b).start()

    def outer(t, carry):
        for b in range(SC_NB):
            k = SC_NB * t + b
            g_desc(k, b).wait()
            w_desc(k, b).start()
            w_desc(k, b).wait()
            g_desc(k + SC_NB, b).start()
        return carry

    lax.fori_loop(0, SC_NCH // SC_NB - 1, outer, 0)
    for b in range(SC_NB):
        k = SC_NCH - SC_NB + b
        g_desc(k, b).wait()
        w_desc(k, b).start()
        w_desc(k, b).wait()


@functools.partial(
    pl.kernel,
    out_type=jax.ShapeDtypeStruct((SC_R, F), jnp.float32),
    mesh=plsc.VectorSubcoreMesh(core_axis_name="c", subcore_axis_name="s"),
    scratch_types=(
        [pltpu.VMEM((SC_PW,), jnp.int32),
         pltpu.VMEM((SC_NB, SC_C, F), jnp.float32)]
        + [pltpu.SemaphoreType.DMA] * (2 * SC_NB)
    ),
)
def _sc_gather(x_hbm, gidx_hbm, out_hbm, idx_all, rows, *sems):
    _sc_gather_body(x_hbm, gidx_hbm, out_hbm, idx_all, rows, *sems)


def _xw_body(f_ref, w_ref, o_ref):
    o_ref[...] = jnp.dot(f_ref[...], w_ref[...], preferred_element_type=jnp.float32)


def _project_x(features2, W0):
    M = features2.shape[0]
    return pl.pallas_call(
        _xw_body,
        grid=(M // TM,),
        in_specs=[
            pl.BlockSpec((TM, F), lambda i: (i, 0)),
            pl.BlockSpec((F, F), lambda i: (0, 0)),
        ],
        out_specs=pl.BlockSpec((TM, F), lambda i: (i, 0)),
        out_shape=jax.ShapeDtypeStruct((M, F), jnp.float32),
    )(features2, W0)


def _fused_body(nbr_ref, rbf_ref, wf1_ref, bf1_ref, wf2_ref, bf2_ref, v_ref,
                emat_ref, w1_ref, b1_ref, w2_ref, b2_ref, out_ref, attn_ref,
                filt_scr):
    cols = []
    for n in range(N):
        r_n = rbf_ref[:, n * G:(n + 1) * G]
        h_n = jnp.tanh(jnp.dot(r_n, wf1_ref[...], preferred_element_type=jnp.float32)
                       + bf1_ref[...])
        f_n = jnp.dot(h_n, wf2_ref[...], preferred_element_type=jnp.float32) + bf2_ref[...]
        filt_n = nbr_ref[n] * f_n
        filt_scr[n] = filt_n
        cols.append(jnp.dot(filt_n, v_ref[...], preferred_element_type=jnp.float32))
    logits = jnp.concatenate(cols, axis=1)
    m = jnp.max(logits, axis=-1, keepdims=True)
    e = jnp.exp(logits - m)
    s = jnp.sum(e, axis=-1, keepdims=True)
    attn = e / s
    attn_ref[...] = attn
    bc = jnp.dot(attn, emat_ref[...], preferred_element_type=jnp.float32)
    conv = jnp.zeros((TA, F), dtype=jnp.float32)
    for n in range(N):
        conv = conv + bc[:, n * F:(n + 1) * F] * filt_scr[n]
    t = jnp.tanh(jnp.dot(conv, w1_ref[...], preferred_element_type=jnp.float32)
                 + b1_ref[...])
    out_ref[...] = jnp.dot(t, w2_ref[...], preferred_element_type=jnp.float32) + b2_ref[...]


def _fused_one(nbr3, rbf2, Wf1, bf1, Wf2, bf2, v, emat, W1, b1, W2, b2):
    full = lambda j: (0, 0)
    out_shape = (
        jax.ShapeDtypeStruct((A, F), jnp.float32),
        jax.ShapeDtypeStruct((A, N), jnp.float32),
    )
    return pl.pallas_call(
        _fused_body,
        grid=(A // TA,),
        in_specs=[
            pl.BlockSpec((N, TA, F), lambda j: (0, j, 0)),
            pl.BlockSpec((TA, N * G), lambda j: (j, 0)),
            pl.BlockSpec((G, F), full),
            pl.BlockSpec((1, F), full),
            pl.BlockSpec((F, F), full),
            pl.BlockSpec((1, F), full),
            pl.BlockSpec((F, 1), full),
            pl.BlockSpec((N, N * F), full),
            pl.BlockSpec((F, F), full),
            pl.BlockSpec((1, F), full),
            pl.BlockSpec((F, F), full),
            pl.BlockSpec((1, F), full),
        ],
        out_specs=(
            pl.BlockSpec((TA, F), lambda j: (j, 0)),
            pl.BlockSpec((TA, N), lambda j: (j, 0)),
        ),
        out_shape=out_shape,
        scratch_shapes=[
            pltpu.VMEM((N, TA, F), jnp.float32),
        ],
    )(nbr3, rbf2, Wf1, bf1, Wf2, bf2, v, emat, W1, b1, W2, b2)


def kernel(features, rbf_expansion, neighbor_list, W0, Wf1, bf1, Wf2, bf2,
           nbr_filter, W1, b1, W2, b2):
    bf1r, bf2r = bf1.reshape(1, F), bf2.reshape(1, F)
    b1r, b2r = b1.reshape(1, F), b2.reshape(1, F)
    emat = jnp.repeat(jnp.eye(N, dtype=jnp.float32), F, axis=1)
    outs, attns = [], []
    for b in range(B):
        x2 = _project_x(features[b], W0)
        gidx = jnp.swapaxes(neighbor_list[b], 0, 1).reshape(-1)
        nbr3 = _sc_gather(x2, gidx).reshape(N, A, F)
        o, at = _fused_one(
            nbr3, rbf_expansion[b].reshape(A, N * G), Wf1, bf1r, Wf2, bf2r,
            nbr_filter, emat, W1, b1r, W2, b2r)
        outs.append(o)
        attns.append(at)
    return (jnp.stack(outs), jnp.stack(attns))
